# Initial kernel scaffold; baseline (speedup 1.0000x reference)
import jax
import jax.numpy as jnp
from jax.experimental import pallas as pl


def kernel(node_embedding, edge_embedding, nbr_idx, nbr_mask, W2, b2, W3, b3, bn_gamma, bn_beta):
    raise NotImplementedError("write your pallas kernel here")


# speedup vs baseline: 6.6306x; 6.6306x over previous
import functools

import jax
import jax.numpy as jnp
from jax import lax
from jax.experimental import pallas as pl
from jax.experimental.pallas import tpu as pltpu
from jax.experimental.pallas import tpu_sc as plsc


B, At, Nbr = 2, 256, 16
N_NODE, N_EDGE = 64, 64
ROWS = B * At * Nbr
ATOMS = B * At
_NC, _NS = 2, 16
_NW = _NC * _NS
_PER_W = ROWS // _NW
_CH = 128


def _dot(a, b):
    return jax.lax.dot_general(
        a, b, (((1,), (0,)), ((), ())),
        precision=jax.lax.Precision.HIGHEST,
        preferred_element_type=jnp.float32)


def _sc_gather_body(table_hbm, idx_hbm, out_hbm,
                    idx_a, idx_b, rows_a, rows_b, sem_a, sem_b):
    wid = lax.axis_index("s") * _NC + lax.axis_index("c")
    base = wid * _PER_W
    pltpu.sync_copy(idx_hbm.at[pl.ds(base, _CH)], idx_a)
    pltpu.sync_copy(idx_hbm.at[pl.ds(base + _CH, _CH)], idx_b)
    ca = pltpu.async_copy(table_hbm.at[idx_a], rows_a, sem_a)
    cb = pltpu.async_copy(table_hbm.at[idx_b], rows_b, sem_b)
    ca.wait()
    pltpu.sync_copy(rows_a, out_hbm.at[pl.ds(base, _CH)])
    cb.wait()
    pltpu.sync_copy(rows_b, out_hbm.at[pl.ds(base + _CH, _CH)])


@functools.cache
def _sc_gather():
    return pl.kernel(
        _sc_gather_body,
        out_type=jax.ShapeDtypeStruct((ROWS, N_NODE), jnp.float32),
        mesh=plsc.VectorSubcoreMesh(core_axis_name="c", subcore_axis_name="s",
                                    num_cores=_NC, num_subcores=_NS),
        scratch_types=[
            pltpu.VMEM((_CH,), jnp.int32),
            pltpu.VMEM((_CH,), jnp.int32),
            pltpu.VMEM((_CH, N_NODE), jnp.float32),
            pltpu.VMEM((_CH, N_NODE), jnp.float32),
            pltpu.SemaphoreType.DMA,
            pltpu.SemaphoreType.DMA,
        ],
        compiler_params=pltpu.CompilerParams(use_tc_tiling_on_sc=False),
    )


def _t1_body(node_ref, nj_ref, edge_ref, mask_ref,
             w2t_ref, w3ni_ref, w3nj_ref, w3nk_ref, w3eij_ref, w3ejk_ref,
             b2_ref, b3_ref,
             u_ref, t_ref, base_ref):
    node = node_ref[...]
    nj = nj_ref[...]
    edge = edge_ref[...]
    m = mask_ref[...]
    njm = nj * m

    prod = (node[:, None, :] * njm.reshape(ATOMS, Nbr, N_NODE)).reshape(ROWS, N_NODE)
    c2 = _dot(prod, w2t_ref[...]) + b2_ref[...]
    base_ref[...] = edge + jax.nn.sigmoid(c2[:, :N_EDGE]) * jnp.tanh(c2[:, N_EDGE:])

    a_i = _dot(node, w3ni_ref[...])
    u = _dot(njm, w3nj_ref[...]) + _dot(edge, w3eij_ref[...]) + b3_ref[...]
    u_ref[...] = (u.reshape(ATOMS, Nbr, 128) + a_i[:, None, :]).reshape(ROWS, 128)
    t_ref[...] = _dot(nj, w3nk_ref[...]) + _dot(edge, w3ejk_ref[...])


def _t1_call(node, nj, edge, mask, w2t, w3ni, w3nj, w3nk, w3eij, w3ejk, b2, b3):
    return pl.pallas_call(
        _t1_body,
        out_shape=(
            jax.ShapeDtypeStruct((ROWS, 128), jnp.float32),
            jax.ShapeDtypeStruct((ROWS, 128), jnp.float32),
            jax.ShapeDtypeStruct((ROWS, N_EDGE), jnp.float32),
        ),
    )(node, nj, edge, mask, w2t, w3ni, w3nj, w3nk, w3eij, w3ejk, b2, b3)


def _t2_body(idx_ref, t_ref, u_ref, m_ref, three_ref):
    p = pl.program_id(0)
    blocks = []
    mrows = []
    for j in range(Nbr):
        a = idx_ref[p, j]
        blocks.append(t_ref[a])
        mrows.append(m_ref[pl.ds(a, 1), :])
    x = jnp.stack(blocks, axis=0)
    mg = jnp.concatenate(mrows, axis=0)
    c = x + u_ref[0][:, None, :]
    s = jax.nn.sigmoid(c)
    th = jnp.tanh(c)
    v = s[:, :, :N_EDGE] * th[:, :, N_EDGE:] * mg[:, :, None]
    three_ref[0] = jnp.sum(v, axis=1)


def _t2_call(idx2, t3, u3, m2):
    return pl.pallas_call(
        _t2_body,
        grid=(ATOMS,),
        in_specs=[
            pl.BlockSpec(memory_space=pltpu.SMEM),
            pl.BlockSpec((ATOMS, Nbr, 128), lambda p: (0, 0, 0)),
            pl.BlockSpec((1, Nbr, 128), lambda p: (p, 0, 0)),
            pl.BlockSpec((ATOMS, Nbr), lambda p: (0, 0)),
        ],
        out_specs=pl.BlockSpec((1, Nbr, N_EDGE), lambda p: (p, 0, 0)),
        out_shape=jax.ShapeDtypeStruct((ATOMS, Nbr, N_EDGE), jnp.float32),
    )(idx2, t3, u3, m2)


def _t3_body(three_ref, base_ref, gamma_ref, beta_ref, out_ref):
    th = three_ref[...]
    mean = jnp.mean(th, axis=0, keepdims=True)
    cent = th - mean
    var = jnp.mean(cent * cent, axis=0, keepdims=True)
    normed = cent * jax.lax.rsqrt(var + 1e-5) * gamma_ref[...] + beta_ref[...]
    out_ref[...] = jnp.tanh(base_ref[...] + normed)


def _t3_call(three, base, gamma, beta):
    return pl.pallas_call(
        _t3_body,
        out_shape=jax.ShapeDtypeStruct((ROWS, N_EDGE), jnp.float32),
    )(three, base, gamma, beta)


def kernel(node_embedding, edge_embedding, nbr_idx, nbr_mask,
           W2, b2, W3, b3, bn_gamma, bn_beta):
    assert node_embedding.shape == (B, At, N_NODE)
    assert edge_embedding.shape == (B, At, Nbr, N_EDGE)

    node_flat = node_embedding.reshape(ATOMS, N_NODE)
    edge_flat = edge_embedding.reshape(ROWS, N_EDGE)
    mask_flat = nbr_mask.reshape(ROWS, 1)
    offs = (jnp.arange(B, dtype=jnp.int32) * At)[:, None, None]
    g_idx = (nbr_idx + offs).reshape(ROWS)

    w2t = W2.T
    w3t = W3.T
    w3ni, w3nj, w3nk = w3t[0:64], w3t[64:128], w3t[128:192]
    w3eij, w3ejk = w3t[192:256], w3t[256:320]
    b2r = b2.reshape(1, 128)
    b3r = b3.reshape(1, 128)

    nj = _sc_gather()(node_flat, g_idx)

    u, t, base = _t1_call(node_flat, nj, edge_flat, mask_flat,
                          w2t, w3ni, w3nj, w3nk, w3eij, w3ejk, b2r, b3r)

    three = _t2_call(g_idx.reshape(ATOMS, Nbr),
                     t.reshape(ATOMS, Nbr, 128),
                     u.reshape(ATOMS, Nbr, 128),
                     nbr_mask.reshape(ATOMS, Nbr))

    out = _t3_call(three.reshape(ROWS, N_EDGE), base,
                   bn_gamma.reshape(1, N_EDGE), bn_beta.reshape(1, N_EDGE))
    return out.reshape(B, At, Nbr, N_EDGE)
